# baseline (speedup 1.0000x reference)
import jax
import jax.numpy as jnp
from jax import lax
from jax.experimental import pallas as pl
from jax.experimental.pallas import tpu as pltpu
from jax.experimental.pallas import tpu_sc as plsc

NUM_U = 5000
N = 10000
H = 128
E = 320000
NC = 2
NS = 16
NW = NC * NS
EW = E // NW
K = 128
NCH = 79
EWP = NCH * K
NCHP = 80
NP = 10240
RPT = NP // NS
CW = 128

_MESH = dict(core_axis_name="c", subcore_axis_name="s")


def _worker_prelude(rowidx_v):
    cid = lax.axis_index("c")
    sid = lax.axis_index("s")
    wid = sid * NC + cid
    base = wid * NCHP
    for i in range(NCHP // 16):
        rowidx_v[pl.ds(i * 16, 16)] = base + i * 16 + lax.iota(jnp.int32, 16)
    return cid, sid


def _unpack_chunk(pk_v, schunk, dchunk, slot, j):
    for c in range(K // 16):
        pk = pk_v[j, pl.ds(c * 16, 16)]
        schunk[slot, pl.ds(c * 16, 16)] = lax.shift_right_logical(pk, 16)
        dchunk[slot, pl.ds(c * 16, 16)] = lax.bitwise_and(pk, 0xFFFF)


GS = 4
KG = K // GS


def _issue_gather(x_hbm, schunk, slot, gb, sems):
    for h in range(GS):
        pltpu.async_copy(
            x_hbm.at[schunk.at[slot, pl.ds(h * KG, KG)]],
            gb.at[pl.ds(h * KG, KG)], sems[h])


def _wait_gather(x_hbm, schunk, slot, gb, sems):
    for h in range(GS):
        pltpu.make_async_copy(
            x_hbm.at[schunk.at[slot, pl.ds(h * KG, KG)]],
            gb.at[pl.ds(h * KG, KG)], sems[h]).wait()


def _sc_agg_body(x_hbm, pk_hbm, zrow_hbm, aggp_hbm,
                 rowidx_v, pk_v, schunk, dchunk, gb0, gb1,
                 sem00, sem01, sem02, sem03, sem10, sem11, sem12, sem13,
                 agg_sh):
    sems0 = (sem00, sem01, sem02, sem03)
    sems1 = (sem10, sem11, sem12, sem13)
    cid, sid = _worker_prelude(rowidx_v)
    rb = sid * RPT

    pltpu.async_copy(pk_hbm.at[rowidx_v], pk_v, sem00)
    pltpu.sync_copy(zrow_hbm, agg_sh.at[pl.ds(rb, RPT)])
    pltpu.make_async_copy(pk_hbm.at[rowidx_v], pk_v, sem00).wait()
    plsc.subcore_barrier()

    _unpack_chunk(pk_v, schunk, dchunk, 0, 0)
    _issue_gather(x_hbm, schunk, 0, gb0, sems0)

    def pair(p, carry):
        j0 = 2 * p
        _unpack_chunk(pk_v, schunk, dchunk, 1, j0 + 1)
        _issue_gather(x_hbm, schunk, 1, gb1, sems1)
        _wait_gather(x_hbm, schunk, 0, gb0, sems0)
        pltpu.sync_copy(gb0, agg_sh.at[dchunk.at[0]], add=True)
        _unpack_chunk(pk_v, schunk, dchunk, 0, j0 + 2)
        _issue_gather(x_hbm, schunk, 0, gb0, sems0)
        _wait_gather(x_hbm, schunk, 1, gb1, sems1)
        pltpu.sync_copy(gb1, agg_sh.at[dchunk.at[1]], add=True)
        return carry

    lax.fori_loop(0, (NCH - 1) // 2, pair, 0)
    _wait_gather(x_hbm, schunk, 0, gb0, sems0)
    pltpu.sync_copy(gb0, agg_sh.at[dchunk.at[0]], add=True)
    plsc.subcore_barrier()

    pltpu.sync_copy(agg_sh.at[pl.ds(rb, RPT)],
                    aggp_hbm.at[cid, pl.ds(rb, RPT)])


_sc_agg = pl.kernel(
    _sc_agg_body,
    out_type=[jax.ShapeDtypeStruct((NC, NP, H), jnp.float32)],
    mesh=plsc.VectorSubcoreMesh(**_MESH),
    scratch_types=[
        pltpu.VMEM((NCHP,), jnp.int32),
        pltpu.VMEM((NCHP, K), jnp.int32),
        pltpu.VMEM((2, K), jnp.int32),
        pltpu.VMEM((2, K), jnp.int32),
        pltpu.VMEM((K, H), jnp.float32),
        pltpu.VMEM((K, H), jnp.float32),
        pltpu.SemaphoreType.DMA,
        pltpu.SemaphoreType.DMA,
        pltpu.SemaphoreType.DMA,
        pltpu.SemaphoreType.DMA,
        pltpu.SemaphoreType.DMA,
        pltpu.SemaphoreType.DMA,
        pltpu.SemaphoreType.DMA,
        pltpu.SemaphoreType.DMA,
        pltpu.VMEM_SHARED((NP, H), jnp.float32),
    ],
)


def _sc_cnt_body(pk_hbm, zcnt_hbm, ones_hbm, cntp_hbm,
                 rowidx_v, pk_v, dchunk, ones_v, sem0, cnt_sh):
    cid, sid = _worker_prelude(rowidx_v)
    rb = sid * RPT

    pltpu.async_copy(pk_hbm.at[rowidx_v], pk_v, sem0)
    pltpu.sync_copy(zcnt_hbm, cnt_sh.at[pl.ds(rb, RPT)])
    pltpu.sync_copy(ones_hbm, ones_v)
    pltpu.make_async_copy(pk_hbm.at[rowidx_v], pk_v, sem0).wait()
    plsc.subcore_barrier()

    def step(j, carry):
        for c in range(K // 16):
            pk = pk_v[j, pl.ds(c * 16, 16)]
            dchunk[0, pl.ds(c * 16, 16)] = lax.bitwise_and(pk, 0xFFFF)
        pltpu.sync_copy(ones_v, cnt_sh.at[dchunk.at[0]], add=True)
        return carry

    lax.fori_loop(0, NCH, step, 0)
    plsc.subcore_barrier()

    pltpu.sync_copy(cnt_sh.at[pl.ds(rb, RPT)],
                    cntp_hbm.at[cid, pl.ds(rb, RPT)])


_sc_cnt = pl.kernel(
    _sc_cnt_body,
    out_type=[jax.ShapeDtypeStruct((NC, NP, CW), jnp.float32)],
    mesh=plsc.VectorSubcoreMesh(**_MESH),
    scratch_types=[
        pltpu.VMEM((NCHP,), jnp.int32),
        pltpu.VMEM((NCHP, K), jnp.int32),
        pltpu.VMEM((1, K), jnp.int32),
        pltpu.VMEM((K, CW), jnp.float32),
        pltpu.SemaphoreType.DMA,
        pltpu.VMEM_SHARED((NP, CW), jnp.float32),
    ],
)

BR = 1000


def _tc_body(aggp_ref, cntp_ref, x_ref, wl_ref, b_ref, wr_ref, o_ref):
    a = aggp_ref[0] + aggp_ref[1]
    c = jnp.maximum(cntp_ref[0, :, 0] + cntp_ref[1, :, 0], 1.0)
    agg = a / c[:, None]
    h = (jnp.dot(agg, wl_ref[...], preferred_element_type=jnp.float32)
         + b_ref[...]
         + jnp.dot(x_ref[...], wr_ref[...], preferred_element_type=jnp.float32))
    o_ref[...] = jnp.maximum(h, 0.0)


def _tc_layer(aggp, cntp, x, wl_t, b, wr_t):
    return pl.pallas_call(
        _tc_body,
        grid=(N // BR,),
        in_specs=[
            pl.BlockSpec((NC, BR, H), lambda i: (0, i, 0)),
            pl.BlockSpec((NC, BR, CW), lambda i: (0, i, 0)),
            pl.BlockSpec((BR, H), lambda i: (i, 0)),
            pl.BlockSpec((H, H), lambda i: (0, 0)),
            pl.BlockSpec((1, H), lambda i: (0, 0)),
            pl.BlockSpec((H, H), lambda i: (0, 0)),
        ],
        out_specs=pl.BlockSpec((BR, H), lambda i: (i, 0)),
        out_shape=jax.ShapeDtypeStruct((N, H), jnp.float32),
    )(aggp, cntp, x, wl_t, b, wr_t)


def _pack_idx(edge_index):
    src = edge_index[0].reshape(NW, EW)
    dst = edge_index[1].reshape(NW, EW)
    src = jnp.pad(src, ((0, 0), (0, EWP - EW)), constant_values=0)
    dst = jnp.pad(dst, ((0, 0), (0, EWP - EW)), constant_values=N)
    pk = jnp.left_shift(src, 16) | dst
    pk = pk.reshape(NW, NCH, K)
    pk = jnp.pad(pk, ((0, 0), (0, NCHP - NCH), (0, 0)))
    return pk.reshape(NW * NCHP, K)


def kernel(edge_index, user_emb, item_emb, W1_l, b1, W1_r, W2_l, b2, W2_r):
    x = jnp.concatenate([user_emb, item_emb], axis=0)
    pk = _pack_idx(edge_index)
    zrow = jnp.zeros((RPT, H), jnp.float32)
    zcnt = jnp.zeros((RPT, CW), jnp.float32)
    ones = jnp.ones((K, CW), jnp.float32)

    (cntp,) = _sc_cnt(pk, zcnt, ones)
    (aggp1,) = _sc_agg(x, pk, zrow)
    h1 = _tc_layer(aggp1, cntp, x, W1_l.T, b1.reshape(1, H), W1_r.T)
    (aggp2,) = _sc_agg(h1, pk, zrow)
    h2 = _tc_layer(aggp2, cntp, h1, W2_l.T, b2.reshape(1, H), W2_r.T)
    return h2[:NUM_U], h2[NUM_U:]


# speedup vs baseline: 1.0040x; 1.0040x over previous
import jax
import jax.numpy as jnp
from jax import lax
from jax.experimental import pallas as pl
from jax.experimental.pallas import tpu as pltpu
from jax.experimental.pallas import tpu_sc as plsc

NUM_U = 5000
N = 10000
H = 128
E = 320000
NC = 2
NS = 16
NW = NC * NS
EW = E // NW
K = 128
NCH = 79
EWP = NCH * K
NCHP = 80
NP = 10240
RPT = NP // NS
CW = 128

_MESH = dict(core_axis_name="c", subcore_axis_name="s")


def _worker_prelude(rowidx_v):
    cid = lax.axis_index("c")
    sid = lax.axis_index("s")
    wid = sid * NC + cid
    base = wid * NCHP
    for i in range(NCHP // 16):
        rowidx_v[pl.ds(i * 16, 16)] = base + i * 16 + lax.iota(jnp.int32, 16)
    return cid, sid


def _unpack_chunk(pk_v, schunk, dchunk, slot, j):
    for c in range(K // 16):
        pk = pk_v[j, pl.ds(c * 16, 16)]
        schunk[slot, pl.ds(c * 16, 16)] = lax.shift_right_logical(pk, 16)
        dchunk[slot, pl.ds(c * 16, 16)] = lax.bitwise_and(pk, 0xFFFF)


GS = 1
KG = K // GS


def _issue_gather(x_hbm, schunk, slot, gb, sems):
    for h in range(GS):
        pltpu.async_copy(
            x_hbm.at[schunk.at[slot, pl.ds(h * KG, KG)]],
            gb.at[pl.ds(h * KG, KG)], sems[h])


def _wait_gather(x_hbm, schunk, slot, gb, sems):
    for h in range(GS):
        pltpu.make_async_copy(
            x_hbm.at[schunk.at[slot, pl.ds(h * KG, KG)]],
            gb.at[pl.ds(h * KG, KG)], sems[h]).wait()


def _sc_agg_body(x_hbm, pk_hbm, zrow_hbm, aggp_hbm,
                 rowidx_v, pk_v, schunk, dchunk, gb0, gb1,
                 sem00, sem01, sem02, sem03, sem10, sem11, sem12, sem13,
                 agg_sh):
    sems0 = (sem00, sem01, sem02, sem03)
    sems1 = (sem10, sem11, sem12, sem13)
    cid, sid = _worker_prelude(rowidx_v)
    rb = sid * RPT

    pltpu.async_copy(pk_hbm.at[rowidx_v], pk_v, sem00)
    pltpu.sync_copy(zrow_hbm, agg_sh.at[pl.ds(rb, RPT)])
    pltpu.make_async_copy(pk_hbm.at[rowidx_v], pk_v, sem00).wait()
    plsc.subcore_barrier()

    _unpack_chunk(pk_v, schunk, dchunk, 0, 0)
    _issue_gather(x_hbm, schunk, 0, gb0, sems0)

    def pair(p, carry):
        j0 = 2 * p
        _unpack_chunk(pk_v, schunk, dchunk, 1, j0 + 1)
        _issue_gather(x_hbm, schunk, 1, gb1, sems1)
        _wait_gather(x_hbm, schunk, 0, gb0, sems0)
        pltpu.sync_copy(gb0, agg_sh.at[dchunk.at[0]], add=True)
        _unpack_chunk(pk_v, schunk, dchunk, 0, j0 + 2)
        _issue_gather(x_hbm, schunk, 0, gb0, sems0)
        _wait_gather(x_hbm, schunk, 1, gb1, sems1)
        pltpu.sync_copy(gb1, agg_sh.at[dchunk.at[1]], add=True)
        return carry

    lax.fori_loop(0, (NCH - 1) // 2, pair, 0)
    _wait_gather(x_hbm, schunk, 0, gb0, sems0)
    pltpu.sync_copy(gb0, agg_sh.at[dchunk.at[0]], add=True)
    plsc.subcore_barrier()

    pltpu.sync_copy(agg_sh.at[pl.ds(rb, RPT)],
                    aggp_hbm.at[cid, pl.ds(rb, RPT)])


_sc_agg = pl.kernel(
    _sc_agg_body,
    out_type=[jax.ShapeDtypeStruct((NC, NP, H), jnp.float32)],
    mesh=plsc.VectorSubcoreMesh(**_MESH),
    scratch_types=[
        pltpu.VMEM((NCHP,), jnp.int32),
        pltpu.VMEM((NCHP, K), jnp.int32),
        pltpu.VMEM((2, K), jnp.int32),
        pltpu.VMEM((2, K), jnp.int32),
        pltpu.VMEM((K, H), jnp.float32),
        pltpu.VMEM((K, H), jnp.float32),
        pltpu.SemaphoreType.DMA,
        pltpu.SemaphoreType.DMA,
        pltpu.SemaphoreType.DMA,
        pltpu.SemaphoreType.DMA,
        pltpu.SemaphoreType.DMA,
        pltpu.SemaphoreType.DMA,
        pltpu.SemaphoreType.DMA,
        pltpu.SemaphoreType.DMA,
        pltpu.VMEM_SHARED((NP, H), jnp.float32),
    ],
)


def _sc_cnt_body(pk_hbm, zcnt_hbm, ones_hbm, cntp_hbm,
                 rowidx_v, pk_v, dchunk, ones_v, sem0, cnt_sh):
    cid, sid = _worker_prelude(rowidx_v)
    rb = sid * RPT

    pltpu.async_copy(pk_hbm.at[rowidx_v], pk_v, sem0)
    pltpu.sync_copy(zcnt_hbm, cnt_sh.at[pl.ds(rb, RPT)])
    pltpu.sync_copy(ones_hbm, ones_v)
    pltpu.make_async_copy(pk_hbm.at[rowidx_v], pk_v, sem0).wait()
    plsc.subcore_barrier()

    def step(j, carry):
        for c in range(K // 16):
            pk = pk_v[j, pl.ds(c * 16, 16)]
            dchunk[0, pl.ds(c * 16, 16)] = lax.bitwise_and(pk, 0xFFFF)
        pltpu.sync_copy(ones_v, cnt_sh.at[dchunk.at[0]], add=True)
        return carry

    lax.fori_loop(0, NCH, step, 0)
    plsc.subcore_barrier()

    pltpu.sync_copy(cnt_sh.at[pl.ds(rb, RPT)],
                    cntp_hbm.at[cid, pl.ds(rb, RPT)])


_sc_cnt = pl.kernel(
    _sc_cnt_body,
    out_type=[jax.ShapeDtypeStruct((NC, NP, CW), jnp.float32)],
    mesh=plsc.VectorSubcoreMesh(**_MESH),
    scratch_types=[
        pltpu.VMEM((NCHP,), jnp.int32),
        pltpu.VMEM((NCHP, K), jnp.int32),
        pltpu.VMEM((1, K), jnp.int32),
        pltpu.VMEM((K, CW), jnp.float32),
        pltpu.SemaphoreType.DMA,
        pltpu.VMEM_SHARED((NP, CW), jnp.float32),
    ],
)

BR = 1000


def _tc_body(aggp_ref, cntp_ref, x_ref, wl_ref, b_ref, wr_ref, o_ref):
    a = aggp_ref[0] + aggp_ref[1]
    c = jnp.maximum(cntp_ref[0, :, 0] + cntp_ref[1, :, 0], 1.0)
    agg = a / c[:, None]
    h = (jnp.dot(agg, wl_ref[...], preferred_element_type=jnp.float32)
         + b_ref[...]
         + jnp.dot(x_ref[...], wr_ref[...], preferred_element_type=jnp.float32))
    o_ref[...] = jnp.maximum(h, 0.0)


def _tc_layer(aggp, cntp, x, wl_t, b, wr_t):
    return pl.pallas_call(
        _tc_body,
        grid=(N // BR,),
        in_specs=[
            pl.BlockSpec((NC, BR, H), lambda i: (0, i, 0)),
            pl.BlockSpec((NC, BR, CW), lambda i: (0, i, 0)),
            pl.BlockSpec((BR, H), lambda i: (i, 0)),
            pl.BlockSpec((H, H), lambda i: (0, 0)),
            pl.BlockSpec((1, H), lambda i: (0, 0)),
            pl.BlockSpec((H, H), lambda i: (0, 0)),
        ],
        out_specs=pl.BlockSpec((BR, H), lambda i: (i, 0)),
        out_shape=jax.ShapeDtypeStruct((N, H), jnp.float32),
    )(aggp, cntp, x, wl_t, b, wr_t)


def _pack_idx(edge_index):
    src = edge_index[0].reshape(NW, EW)
    dst = edge_index[1].reshape(NW, EW)
    src = jnp.pad(src, ((0, 0), (0, EWP - EW)), constant_values=0)
    dst = jnp.pad(dst, ((0, 0), (0, EWP - EW)), constant_values=N)
    pk = jnp.left_shift(src, 16) | dst
    pk = pk.reshape(NW, NCH, K)
    pk = jnp.pad(pk, ((0, 0), (0, NCHP - NCH), (0, 0)))
    return pk.reshape(NW * NCHP, K)


def kernel(edge_index, user_emb, item_emb, W1_l, b1, W1_r, W2_l, b2, W2_r):
    x = jnp.concatenate([user_emb, item_emb], axis=0)
    pk = _pack_idx(edge_index)
    zrow = jnp.zeros((RPT, H), jnp.float32)
    zcnt = jnp.zeros((RPT, CW), jnp.float32)
    ones = jnp.ones((K, CW), jnp.float32)

    (cntp,) = _sc_cnt(pk, zcnt, ones)
    (aggp1,) = _sc_agg(x, pk, zrow)
    h1 = _tc_layer(aggp1, cntp, x, W1_l.T, b1.reshape(1, H), W1_r.T)
    (aggp2,) = _sc_agg(h1, pk, zrow)
    h2 = _tc_layer(aggp2, cntp, h1, W2_l.T, b2.reshape(1, H), W2_r.T)
    return h2[:NUM_U], h2[NUM_U:]


# speedup vs baseline: 1.0143x; 1.0103x over previous
import jax
import jax.numpy as jnp
from jax import lax
from jax.experimental import pallas as pl
from jax.experimental.pallas import tpu as pltpu
from jax.experimental.pallas import tpu_sc as plsc

NUM_U = 5000
N = 10000
H = 128
E = 320000
NC = 2
NS = 16
NW = NC * NS
EW = E // NW
K = 128
NCH = 79
EWP = NCH * K
NCHP = 80
NP = 10240
RPT = NP // NS
CW = 128

_MESH = dict(core_axis_name="c", subcore_axis_name="s")


def _worker_prelude(rowidx_v):
    cid = lax.axis_index("c")
    sid = lax.axis_index("s")
    wid = sid * NC + cid
    base = wid * NCHP
    for i in range(NCHP // 16):
        rowidx_v[pl.ds(i * 16, 16)] = base + i * 16 + lax.iota(jnp.int32, 16)
    return cid, sid


def _unpack_chunk(pk_v, schunk, dchunk, slot, j):
    for c in range(K // 16):
        pk = pk_v[j, pl.ds(c * 16, 16)]
        schunk[slot, pl.ds(c * 16, 16)] = lax.shift_right_logical(pk, 16)
        dchunk[slot, pl.ds(c * 16, 16)] = lax.bitwise_and(pk, 0xFFFF)


GS = 1
KG = K // GS


def _issue_gather(x_hbm, schunk, slot, gb, sems):
    for h in range(GS):
        pltpu.async_copy(
            x_hbm.at[schunk.at[slot, pl.ds(h * KG, KG)]],
            gb.at[pl.ds(h * KG, KG)], sems[h])


def _wait_gather(x_hbm, schunk, slot, gb, sems):
    for h in range(GS):
        pltpu.make_async_copy(
            x_hbm.at[schunk.at[slot, pl.ds(h * KG, KG)]],
            gb.at[pl.ds(h * KG, KG)], sems[h]).wait()


def _sc_agg_body(x_hbm, pk_hbm, zrow_hbm, aggp_hbm,
                 rowidx_v, pk_v, schunk, dchunk, gb0, gb1,
                 sem00, sem01, sem02, sem03, sem10, sem11, sem12, sem13,
                 agg_sh):
    sems0 = (sem00, sem01, sem02, sem03)
    sems1 = (sem10, sem11, sem12, sem13)
    cid, sid = _worker_prelude(rowidx_v)
    rb = sid * RPT

    pltpu.async_copy(pk_hbm.at[rowidx_v], pk_v, sem00)
    pltpu.sync_copy(zrow_hbm, agg_sh.at[pl.ds(rb, RPT)])
    pltpu.make_async_copy(pk_hbm.at[rowidx_v], pk_v, sem00).wait()
    plsc.subcore_barrier()

    _unpack_chunk(pk_v, schunk, dchunk, 0, 0)
    _issue_gather(x_hbm, schunk, 0, gb0, sems0)

    def pair(p, carry):
        j0 = 2 * p
        _unpack_chunk(pk_v, schunk, dchunk, 1, j0 + 1)
        _issue_gather(x_hbm, schunk, 1, gb1, sems1)
        _wait_gather(x_hbm, schunk, 0, gb0, sems0)
        pltpu.sync_copy(gb0, agg_sh.at[dchunk.at[0]], add=True)
        _unpack_chunk(pk_v, schunk, dchunk, 0, j0 + 2)
        _issue_gather(x_hbm, schunk, 0, gb0, sems0)
        _wait_gather(x_hbm, schunk, 1, gb1, sems1)
        pltpu.sync_copy(gb1, agg_sh.at[dchunk.at[1]], add=True)
        return carry

    lax.fori_loop(0, (NCH - 1) // 2, pair, 0)
    _wait_gather(x_hbm, schunk, 0, gb0, sems0)
    pltpu.sync_copy(gb0, agg_sh.at[dchunk.at[0]], add=True)
    plsc.subcore_barrier()

    pltpu.sync_copy(agg_sh.at[pl.ds(rb, RPT)],
                    aggp_hbm.at[cid, pl.ds(rb, RPT)])


_sc_agg = pl.kernel(
    _sc_agg_body,
    out_type=[jax.ShapeDtypeStruct((NC, NP, H), jnp.float32)],
    mesh=plsc.VectorSubcoreMesh(**_MESH),
    scratch_types=[
        pltpu.VMEM((NCHP,), jnp.int32),
        pltpu.VMEM((NCHP, K), jnp.int32),
        pltpu.VMEM((2, K), jnp.int32),
        pltpu.VMEM((2, K), jnp.int32),
        pltpu.VMEM((K, H), jnp.float32),
        pltpu.VMEM((K, H), jnp.float32),
        pltpu.SemaphoreType.DMA,
        pltpu.SemaphoreType.DMA,
        pltpu.SemaphoreType.DMA,
        pltpu.SemaphoreType.DMA,
        pltpu.SemaphoreType.DMA,
        pltpu.SemaphoreType.DMA,
        pltpu.SemaphoreType.DMA,
        pltpu.SemaphoreType.DMA,
        pltpu.VMEM_SHARED((NP, H), jnp.float32),
    ],
)


def _sc_agg_cnt_body(x_hbm, pk_hbm, zrow_hbm, ones_hbm, aggp_hbm, cntp_hbm,
                     rowidx_v, pk_v, schunk, dchunk, gb0, gb1,
                     sem00, sem01, sem02, sem03, sem10, sem11, sem12, sem13,
                     agg_sh):
    sems0 = (sem00, sem01, sem02, sem03)
    sems1 = (sem10, sem11, sem12, sem13)
    cid, sid = _worker_prelude(rowidx_v)
    rb = sid * RPT

    pltpu.async_copy(pk_hbm.at[rowidx_v], pk_v, sem00)
    pltpu.sync_copy(zrow_hbm, agg_sh.at[pl.ds(rb, RPT)])
    pltpu.sync_copy(ones_hbm, gb1)
    pltpu.make_async_copy(pk_hbm.at[rowidx_v], pk_v, sem00).wait()
    plsc.subcore_barrier()

    _unpack_chunk(pk_v, schunk, dchunk, 0, 0)
    _issue_gather(x_hbm, schunk, 0, gb0, sems0)

    def cstep(j, carry):
        for c in range(K // 16):
            pk = pk_v[j, pl.ds(c * 16, 16)]
            dchunk[1, pl.ds(c * 16, 16)] = lax.bitwise_and(pk, 0xFFFF)
        pltpu.sync_copy(gb1, agg_sh.at[dchunk.at[1]], add=True)
        return carry

    lax.fori_loop(0, NCH, cstep, 0)
    plsc.subcore_barrier()
    pltpu.sync_copy(agg_sh.at[pl.ds(rb, RPT)],
                    cntp_hbm.at[cid, pl.ds(rb, RPT)])
    pltpu.sync_copy(zrow_hbm, agg_sh.at[pl.ds(rb, RPT)])
    plsc.subcore_barrier()

    def pair(p, carry):
        j0 = 2 * p
        _unpack_chunk(pk_v, schunk, dchunk, 1, j0 + 1)
        _issue_gather(x_hbm, schunk, 1, gb1, sems1)
        _wait_gather(x_hbm, schunk, 0, gb0, sems0)
        pltpu.sync_copy(gb0, agg_sh.at[dchunk.at[0]], add=True)
        _unpack_chunk(pk_v, schunk, dchunk, 0, j0 + 2)
        _issue_gather(x_hbm, schunk, 0, gb0, sems0)
        _wait_gather(x_hbm, schunk, 1, gb1, sems1)
        pltpu.sync_copy(gb1, agg_sh.at[dchunk.at[1]], add=True)
        return carry

    lax.fori_loop(0, (NCH - 1) // 2, pair, 0)
    _wait_gather(x_hbm, schunk, 0, gb0, sems0)
    pltpu.sync_copy(gb0, agg_sh.at[dchunk.at[0]], add=True)
    plsc.subcore_barrier()

    pltpu.sync_copy(agg_sh.at[pl.ds(rb, RPT)],
                    aggp_hbm.at[cid, pl.ds(rb, RPT)])


_sc_agg_cnt = pl.kernel(
    _sc_agg_cnt_body,
    out_type=[jax.ShapeDtypeStruct((NC, NP, H), jnp.float32),
              jax.ShapeDtypeStruct((NC, NP, CW), jnp.float32)],
    mesh=plsc.VectorSubcoreMesh(**_MESH),
    scratch_types=[
        pltpu.VMEM((NCHP,), jnp.int32),
        pltpu.VMEM((NCHP, K), jnp.int32),
        pltpu.VMEM((2, K), jnp.int32),
        pltpu.VMEM((2, K), jnp.int32),
        pltpu.VMEM((K, H), jnp.float32),
        pltpu.VMEM((K, H), jnp.float32),
        pltpu.SemaphoreType.DMA,
        pltpu.SemaphoreType.DMA,
        pltpu.SemaphoreType.DMA,
        pltpu.SemaphoreType.DMA,
        pltpu.SemaphoreType.DMA,
        pltpu.SemaphoreType.DMA,
        pltpu.SemaphoreType.DMA,
        pltpu.SemaphoreType.DMA,
        pltpu.VMEM_SHARED((NP, H), jnp.float32),
    ],
)

BR = 1000


def _tc_body(aggp_ref, cntp_ref, x_ref, wl_ref, b_ref, wr_ref, o_ref):
    a = aggp_ref[0] + aggp_ref[1]
    c = jnp.maximum(cntp_ref[0, :, 0] + cntp_ref[1, :, 0], 1.0)
    agg = a / c[:, None]
    h = (jnp.dot(agg, wl_ref[...], preferred_element_type=jnp.float32)
         + b_ref[...]
         + jnp.dot(x_ref[...], wr_ref[...], preferred_element_type=jnp.float32))
    o_ref[...] = jnp.maximum(h, 0.0)


def _tc_layer(aggp, cntp, x, wl_t, b, wr_t):
    return pl.pallas_call(
        _tc_body,
        grid=(N // BR,),
        in_specs=[
            pl.BlockSpec((NC, BR, H), lambda i: (0, i, 0)),
            pl.BlockSpec((NC, BR, CW), lambda i: (0, i, 0)),
            pl.BlockSpec((BR, H), lambda i: (i, 0)),
            pl.BlockSpec((H, H), lambda i: (0, 0)),
            pl.BlockSpec((1, H), lambda i: (0, 0)),
            pl.BlockSpec((H, H), lambda i: (0, 0)),
        ],
        out_specs=pl.BlockSpec((BR, H), lambda i: (i, 0)),
        out_shape=jax.ShapeDtypeStruct((N, H), jnp.float32),
    )(aggp, cntp, x, wl_t, b, wr_t)


def _pack_idx(edge_index):
    src = edge_index[0].reshape(NW, EW)
    dst = edge_index[1].reshape(NW, EW)
    src = jnp.pad(src, ((0, 0), (0, EWP - EW)), constant_values=0)
    dst = jnp.pad(dst, ((0, 0), (0, EWP - EW)), constant_values=N)
    pk = jnp.left_shift(src, 16) | dst
    pk = pk.reshape(NW, NCH, K)
    pk = jnp.pad(pk, ((0, 0), (0, NCHP - NCH), (0, 0)))
    return pk.reshape(NW * NCHP, K)


def kernel(edge_index, user_emb, item_emb, W1_l, b1, W1_r, W2_l, b2, W2_r):
    x = jnp.concatenate([user_emb, item_emb], axis=0)
    pk = _pack_idx(edge_index)
    zrow = jnp.zeros((RPT, H), jnp.float32)
    ones = jnp.ones((K, H), jnp.float32)

    aggp1, cntp = _sc_agg_cnt(x, pk, zrow, ones)
    h1 = _tc_layer(aggp1, cntp, x, W1_l.T, b1.reshape(1, H), W1_r.T)
    (aggp2,) = _sc_agg(h1, pk, zrow)
    h2 = _tc_layer(aggp2, cntp, h1, W2_l.T, b2.reshape(1, H), W2_r.T)
    return h2[:NUM_U], h2[NUM_U:]


# speedup vs baseline: 1.0175x; 1.0031x over previous
import jax
import jax.numpy as jnp
from jax import lax
from jax.experimental import pallas as pl
from jax.experimental.pallas import tpu as pltpu
from jax.experimental.pallas import tpu_sc as plsc

NUM_U = 5000
N = 10000
H = 128
E = 320000
NC = 2
NS = 16
NW = NC * NS
EW = E // NW
K = 128
NCH = 79
EWP = NCH * K
NCHP = 80
NP = 10240
RPT = NP // NS
CW = 128

_MESH = dict(core_axis_name="c", subcore_axis_name="s")


def _worker_prelude(rowidx_v):
    cid = lax.axis_index("c")
    sid = lax.axis_index("s")
    wid = sid * NC + cid
    base = wid * NCHP
    for i in range(NCHP // 16):
        rowidx_v[pl.ds(i * 16, 16)] = base + i * 16 + lax.iota(jnp.int32, 16)
    return cid, sid


def _unpack_chunk(pk_v, schunk, dchunk, slot, j):
    for c in range(K // 16):
        pk = pk_v[j, pl.ds(c * 16, 16)]
        schunk[slot, pl.ds(c * 16, 16)] = lax.shift_right_logical(pk, 16)
        dchunk[slot, pl.ds(c * 16, 16)] = lax.bitwise_and(pk, 0xFFFF)


GS = 1
KG = K // GS


def _issue_gather(x_hbm, schunk, slot, gb, sems):
    for h in range(GS):
        pltpu.async_copy(
            x_hbm.at[schunk.at[slot, pl.ds(h * KG, KG)]],
            gb.at[pl.ds(h * KG, KG)], sems[h])


def _wait_gather(x_hbm, schunk, slot, gb, sems):
    for h in range(GS):
        pltpu.make_async_copy(
            x_hbm.at[schunk.at[slot, pl.ds(h * KG, KG)]],
            gb.at[pl.ds(h * KG, KG)], sems[h]).wait()


def _sc_agg_body(x_hbm, pk_hbm, zrow_hbm, aggp_hbm,
                 rowidx_v, pk_v, schunk, dchunk, gb0, gb1,
                 sem00, sem01, sem02, sem03, sem10, sem11, sem12, sem13,
                 agg_sh):
    sems0 = (sem00, sem01, sem02, sem03)
    sems1 = (sem10, sem11, sem12, sem13)
    cid, sid = _worker_prelude(rowidx_v)
    rb = sid * RPT

    pltpu.async_copy(pk_hbm.at[rowidx_v], pk_v, sem00)
    pltpu.sync_copy(zrow_hbm, agg_sh.at[pl.ds(rb, RPT)])
    pltpu.make_async_copy(pk_hbm.at[rowidx_v], pk_v, sem00).wait()
    plsc.subcore_barrier()

    _unpack_chunk(pk_v, schunk, dchunk, 0, 0)
    _issue_gather(x_hbm, schunk, 0, gb0, sems0)

    def pair(p, carry):
        j0 = 2 * p
        _unpack_chunk(pk_v, schunk, dchunk, 1, j0 + 1)
        _issue_gather(x_hbm, schunk, 1, gb1, sems1)
        _wait_gather(x_hbm, schunk, 0, gb0, sems0)
        pltpu.sync_copy(gb0, agg_sh.at[dchunk.at[0]], add=True)
        _unpack_chunk(pk_v, schunk, dchunk, 0, j0 + 2)
        _issue_gather(x_hbm, schunk, 0, gb0, sems0)
        _wait_gather(x_hbm, schunk, 1, gb1, sems1)
        pltpu.sync_copy(gb1, agg_sh.at[dchunk.at[1]], add=True)
        return carry

    lax.fori_loop(0, (NCH - 1) // 2, pair, 0)
    _wait_gather(x_hbm, schunk, 0, gb0, sems0)
    pltpu.sync_copy(gb0, agg_sh.at[dchunk.at[0]], add=True)
    plsc.subcore_barrier()

    pltpu.sync_copy(agg_sh.at[pl.ds(rb, RPT)],
                    aggp_hbm.at[cid, pl.ds(rb, RPT)])


_sc_agg = pl.kernel(
    _sc_agg_body,
    out_type=[jax.ShapeDtypeStruct((NC, NP, H), jnp.float32)],
    mesh=plsc.VectorSubcoreMesh(**_MESH),
    scratch_types=[
        pltpu.VMEM((NCHP,), jnp.int32),
        pltpu.VMEM((NCHP, K), jnp.int32),
        pltpu.VMEM((2, K), jnp.int32),
        pltpu.VMEM((2, K), jnp.int32),
        pltpu.VMEM((K, H), jnp.float32),
        pltpu.VMEM((K, H), jnp.float32),
        pltpu.SemaphoreType.DMA,
        pltpu.SemaphoreType.DMA,
        pltpu.SemaphoreType.DMA,
        pltpu.SemaphoreType.DMA,
        pltpu.SemaphoreType.DMA,
        pltpu.SemaphoreType.DMA,
        pltpu.SemaphoreType.DMA,
        pltpu.SemaphoreType.DMA,
        pltpu.VMEM_SHARED((NP, H), jnp.float32),
    ],
)


def _sc_agg_cnt_body(x_hbm, pk_hbm, zrow_hbm, ones_hbm, aggp_hbm, cntp_hbm,
                     rowidx_v, pk_v, schunk, dchunk, gb0, gb1,
                     sem00, sem01, sem02, sem03, sem10, sem11, sem12, sem13,
                     agg_sh):
    sems0 = (sem00, sem01, sem02, sem03)
    sems1 = (sem10, sem11, sem12, sem13)
    cid, sid = _worker_prelude(rowidx_v)
    rb = sid * RPT

    pltpu.async_copy(pk_hbm.at[rowidx_v], pk_v, sem00)
    pltpu.sync_copy(zrow_hbm, agg_sh.at[pl.ds(rb, RPT)])
    pltpu.sync_copy(ones_hbm, gb1)
    pltpu.make_async_copy(pk_hbm.at[rowidx_v], pk_v, sem00).wait()
    plsc.subcore_barrier()

    _unpack_chunk(pk_v, schunk, dchunk, 0, 0)
    _issue_gather(x_hbm, schunk, 0, gb0, sems0)

    def _unpack_dst(slot, j):
        for c in range(K // 16):
            pk = pk_v[j, pl.ds(c * 16, 16)]
            dchunk[slot, pl.ds(c * 16, 16)] = lax.bitwise_and(pk, 0xFFFF)

    _unpack_dst(2, 0)
    pltpu.async_copy(gb1, agg_sh.at[dchunk.at[2]], sem02, add=True)
    _unpack_dst(3, 1)
    pltpu.async_copy(gb1, agg_sh.at[dchunk.at[3]], sem03, add=True)

    def cpair(q, carry):
        j0 = 2 * q
        pltpu.make_async_copy(gb1, agg_sh.at[dchunk.at[2]], sem02).wait()
        _unpack_dst(2, j0)
        pltpu.async_copy(gb1, agg_sh.at[dchunk.at[2]], sem02, add=True)
        pltpu.make_async_copy(gb1, agg_sh.at[dchunk.at[3]], sem03).wait()
        _unpack_dst(3, j0 + 1)
        pltpu.async_copy(gb1, agg_sh.at[dchunk.at[3]], sem03, add=True)
        return carry

    lax.fori_loop(1, (NCH - 1) // 2, cpair, 0)
    pltpu.make_async_copy(gb1, agg_sh.at[dchunk.at[2]], sem02).wait()
    _unpack_dst(2, NCH - 1)
    pltpu.async_copy(gb1, agg_sh.at[dchunk.at[2]], sem02, add=True)
    pltpu.make_async_copy(gb1, agg_sh.at[dchunk.at[3]], sem03).wait()
    pltpu.make_async_copy(gb1, agg_sh.at[dchunk.at[2]], sem02).wait()
    plsc.subcore_barrier()
    pltpu.sync_copy(agg_sh.at[pl.ds(rb, RPT)],
                    cntp_hbm.at[cid, pl.ds(rb, RPT)])
    pltpu.sync_copy(zrow_hbm, agg_sh.at[pl.ds(rb, RPT)])
    plsc.subcore_barrier()

    def pair(p, carry):
        j0 = 2 * p
        _unpack_chunk(pk_v, schunk, dchunk, 1, j0 + 1)
        _issue_gather(x_hbm, schunk, 1, gb1, sems1)
        _wait_gather(x_hbm, schunk, 0, gb0, sems0)
        pltpu.sync_copy(gb0, agg_sh.at[dchunk.at[0]], add=True)
        _unpack_chunk(pk_v, schunk, dchunk, 0, j0 + 2)
        _issue_gather(x_hbm, schunk, 0, gb0, sems0)
        _wait_gather(x_hbm, schunk, 1, gb1, sems1)
        pltpu.sync_copy(gb1, agg_sh.at[dchunk.at[1]], add=True)
        return carry

    lax.fori_loop(0, (NCH - 1) // 2, pair, 0)
    _wait_gather(x_hbm, schunk, 0, gb0, sems0)
    pltpu.sync_copy(gb0, agg_sh.at[dchunk.at[0]], add=True)
    plsc.subcore_barrier()

    pltpu.sync_copy(agg_sh.at[pl.ds(rb, RPT)],
                    aggp_hbm.at[cid, pl.ds(rb, RPT)])


_sc_agg_cnt = pl.kernel(
    _sc_agg_cnt_body,
    out_type=[jax.ShapeDtypeStruct((NC, NP, H), jnp.float32),
              jax.ShapeDtypeStruct((NC, NP, CW), jnp.float32)],
    mesh=plsc.VectorSubcoreMesh(**_MESH),
    scratch_types=[
        pltpu.VMEM((NCHP,), jnp.int32),
        pltpu.VMEM((NCHP, K), jnp.int32),
        pltpu.VMEM((2, K), jnp.int32),
        pltpu.VMEM((4, K), jnp.int32),
        pltpu.VMEM((K, H), jnp.float32),
        pltpu.VMEM((K, H), jnp.float32),
        pltpu.SemaphoreType.DMA,
        pltpu.SemaphoreType.DMA,
        pltpu.SemaphoreType.DMA,
        pltpu.SemaphoreType.DMA,
        pltpu.SemaphoreType.DMA,
        pltpu.SemaphoreType.DMA,
        pltpu.SemaphoreType.DMA,
        pltpu.SemaphoreType.DMA,
        pltpu.VMEM_SHARED((NP, H), jnp.float32),
    ],
)

BR = 1000


def _tc_body(aggp_ref, cntp_ref, x_ref, wl_ref, b_ref, wr_ref, o_ref):
    a = aggp_ref[0] + aggp_ref[1]
    c = jnp.maximum(cntp_ref[0, :, 0] + cntp_ref[1, :, 0], 1.0)
    agg = a / c[:, None]
    h = (jnp.dot(agg, wl_ref[...], preferred_element_type=jnp.float32)
         + b_ref[...]
         + jnp.dot(x_ref[...], wr_ref[...], preferred_element_type=jnp.float32))
    o_ref[...] = jnp.maximum(h, 0.0)


def _tc_layer(aggp, cntp, x, wl_t, b, wr_t):
    return pl.pallas_call(
        _tc_body,
        grid=(N // BR,),
        in_specs=[
            pl.BlockSpec((NC, BR, H), lambda i: (0, i, 0)),
            pl.BlockSpec((NC, BR, CW), lambda i: (0, i, 0)),
            pl.BlockSpec((BR, H), lambda i: (i, 0)),
            pl.BlockSpec((H, H), lambda i: (0, 0)),
            pl.BlockSpec((1, H), lambda i: (0, 0)),
            pl.BlockSpec((H, H), lambda i: (0, 0)),
        ],
        out_specs=pl.BlockSpec((BR, H), lambda i: (i, 0)),
        out_shape=jax.ShapeDtypeStruct((N, H), jnp.float32),
    )(aggp, cntp, x, wl_t, b, wr_t)


def _pack_idx(edge_index):
    src = edge_index[0].reshape(NW, EW)
    dst = edge_index[1].reshape(NW, EW)
    src = jnp.pad(src, ((0, 0), (0, EWP - EW)), constant_values=0)
    dst = jnp.pad(dst, ((0, 0), (0, EWP - EW)), constant_values=N)
    pk = jnp.left_shift(src, 16) | dst
    pk = pk.reshape(NW, NCH, K)
    pk = jnp.pad(pk, ((0, 0), (0, NCHP - NCH), (0, 0)))
    return pk.reshape(NW * NCHP, K)


def kernel(edge_index, user_emb, item_emb, W1_l, b1, W1_r, W2_l, b2, W2_r):
    x = jnp.concatenate([user_emb, item_emb], axis=0)
    pk = _pack_idx(edge_index)
    zrow = jnp.zeros((RPT, H), jnp.float32)
    ones = jnp.ones((K, H), jnp.float32)

    aggp1, cntp = _sc_agg_cnt(x, pk, zrow, ones)
    h1 = _tc_layer(aggp1, cntp, x, W1_l.T, b1.reshape(1, H), W1_r.T)
    (aggp2,) = _sc_agg(h1, pk, zrow)
    h2 = _tc_layer(aggp2, cntp, h1, W2_l.T, b2.reshape(1, H), W2_r.T)
    return h2[:NUM_U], h2[NUM_U:]


# speedup vs baseline: 1.0179x; 1.0004x over previous
import jax
import jax.numpy as jnp
from jax import lax
from jax.experimental import pallas as pl
from jax.experimental.pallas import tpu as pltpu
from jax.experimental.pallas import tpu_sc as plsc

NUM_U = 5000
N = 10000
H = 128
E = 320000
NC = 2
NS = 16
NW = NC * NS
EW = E // NW
K = 128
NCH = 79
EWP = NCH * K
NCHP = 80
NP = 10240
RPT = NP // NS
CW = 128

_MESH = dict(core_axis_name="c", subcore_axis_name="s")


def _worker_prelude(rowidx_v):
    cid = lax.axis_index("c")
    sid = lax.axis_index("s")
    wid = sid * NC + cid
    base = wid * NCHP
    for i in range(NCHP // 16):
        rowidx_v[pl.ds(i * 16, 16)] = base + i * 16 + lax.iota(jnp.int32, 16)
    return cid, sid


def _unpack_chunk(pk_v, schunk, dchunk, slot, j):
    for c in range(K // 16):
        pk = pk_v[j, pl.ds(c * 16, 16)]
        schunk[slot, pl.ds(c * 16, 16)] = lax.shift_right_logical(pk, 16)
        dchunk[slot, pl.ds(c * 16, 16)] = lax.bitwise_and(pk, 0xFFFF)


GS = 1
KG = K // GS


def _issue_gather(x_hbm, schunk, slot, gb, sems):
    for h in range(GS):
        pltpu.async_copy(
            x_hbm.at[schunk.at[slot, pl.ds(h * KG, KG)]],
            gb.at[pl.ds(h * KG, KG)], sems[h])


def _wait_gather(x_hbm, schunk, slot, gb, sems):
    for h in range(GS):
        pltpu.make_async_copy(
            x_hbm.at[schunk.at[slot, pl.ds(h * KG, KG)]],
            gb.at[pl.ds(h * KG, KG)], sems[h]).wait()


def _sc_agg_body(x_hbm, pk_hbm, zrow_hbm, aggp_hbm,
                 rowidx_v, pk_v, schunk, dchunk, gb0, gb1,
                 sem00, sem01, sem02, sem03, sem10, sem11, sem12, sem13,
                 agg_sh):
    sems0 = (sem00, sem01, sem02, sem03)
    sems1 = (sem10, sem11, sem12, sem13)
    cid, sid = _worker_prelude(rowidx_v)
    rb = sid * RPT

    pltpu.async_copy(pk_hbm.at[rowidx_v], pk_v, sem00)
    pltpu.sync_copy(zrow_hbm, agg_sh.at[pl.ds(rb, RPT)])
    pltpu.make_async_copy(pk_hbm.at[rowidx_v], pk_v, sem00).wait()
    plsc.subcore_barrier()

    _unpack_chunk(pk_v, schunk, dchunk, 0, 0)
    _issue_gather(x_hbm, schunk, 0, gb0, sems0)
    _unpack_chunk(pk_v, schunk, dchunk, 1, 1)
    _issue_gather(x_hbm, schunk, 1, gb1, sems1)

    def _unpack_sd(s_slot, d_slot, j):
        for c in range(K // 16):
            pk = pk_v[j, pl.ds(c * 16, 16)]
            schunk[s_slot, pl.ds(c * 16, 16)] = lax.shift_right_logical(pk, 16)
            dchunk[d_slot, pl.ds(c * 16, 16)] = lax.bitwise_and(pk, 0xFFFF)

    def _lane(j_next, gb, s_slot, d_cur, d_next, gsems, ssem):
        _wait_gather(x_hbm, schunk, s_slot, gb, gsems)
        pltpu.async_copy(gb, agg_sh.at[dchunk.at[d_cur]], ssem, add=True)
        _unpack_sd(s_slot, d_next, j_next)
        pltpu.make_async_copy(gb, agg_sh.at[dchunk.at[d_cur]], ssem).wait()
        _issue_gather(x_hbm, schunk, s_slot, gb, gsems)

    def quad(p, carry):
        j0 = 4 * p
        _lane(j0 + 2, gb0, 0, 0, 2, sems0, sem01)
        _lane(j0 + 3, gb1, 1, 1, 3, sems1, sem11)
        _lane(j0 + 4, gb0, 0, 2, 0, sems0, sem01)
        _lane(j0 + 5, gb1, 1, 3, 1, sems1, sem11)
        return carry

    lax.fori_loop(0, (NCH - 3) // 4, quad, 0)
    _lane(NCH - 1, gb0, 0, 0, 2, sems0, sem01)
    _wait_gather(x_hbm, schunk, 1, gb1, sems1)
    pltpu.sync_copy(gb1, agg_sh.at[dchunk.at[1]], add=True)
    _wait_gather(x_hbm, schunk, 0, gb0, sems0)
    pltpu.sync_copy(gb0, agg_sh.at[dchunk.at[2]], add=True)
    plsc.subcore_barrier()

    pltpu.sync_copy(agg_sh.at[pl.ds(rb, RPT)],
                    aggp_hbm.at[cid, pl.ds(rb, RPT)])


_sc_agg = pl.kernel(
    _sc_agg_body,
    out_type=[jax.ShapeDtypeStruct((NC, NP, H), jnp.float32)],
    mesh=plsc.VectorSubcoreMesh(**_MESH),
    scratch_types=[
        pltpu.VMEM((NCHP,), jnp.int32),
        pltpu.VMEM((NCHP, K), jnp.int32),
        pltpu.VMEM((2, K), jnp.int32),
        pltpu.VMEM((4, K), jnp.int32),
        pltpu.VMEM((K, H), jnp.float32),
        pltpu.VMEM((K, H), jnp.float32),
        pltpu.SemaphoreType.DMA,
        pltpu.SemaphoreType.DMA,
        pltpu.SemaphoreType.DMA,
        pltpu.SemaphoreType.DMA,
        pltpu.SemaphoreType.DMA,
        pltpu.SemaphoreType.DMA,
        pltpu.SemaphoreType.DMA,
        pltpu.SemaphoreType.DMA,
        pltpu.VMEM_SHARED((NP, H), jnp.float32),
    ],
)


def _sc_agg_cnt_body(x_hbm, pk_hbm, zrow_hbm, ones_hbm, aggp_hbm, cntp_hbm,
                     rowidx_v, pk_v, schunk, dchunk, gb0, gb1,
                     sem00, sem01, sem02, sem03, sem10, sem11, sem12, sem13,
                     agg_sh):
    sems0 = (sem00, sem01, sem02, sem03)
    sems1 = (sem10, sem11, sem12, sem13)
    cid, sid = _worker_prelude(rowidx_v)
    rb = sid * RPT

    pltpu.async_copy(pk_hbm.at[rowidx_v], pk_v, sem00)
    pltpu.sync_copy(zrow_hbm, agg_sh.at[pl.ds(rb, RPT)])
    pltpu.sync_copy(ones_hbm, gb1)
    pltpu.make_async_copy(pk_hbm.at[rowidx_v], pk_v, sem00).wait()
    plsc.subcore_barrier()

    _unpack_chunk(pk_v, schunk, dchunk, 0, 0)
    _issue_gather(x_hbm, schunk, 0, gb0, sems0)

    def _unpack_dst(slot, j):
        for c in range(K // 16):
            pk = pk_v[j, pl.ds(c * 16, 16)]
            dchunk[slot, pl.ds(c * 16, 16)] = lax.bitwise_and(pk, 0xFFFF)

    _unpack_dst(2, 0)
    pltpu.async_copy(gb1, agg_sh.at[dchunk.at[2]], sem02, add=True)
    _unpack_dst(3, 1)
    pltpu.async_copy(gb1, agg_sh.at[dchunk.at[3]], sem03, add=True)

    def cpair(q, carry):
        j0 = 2 * q
        pltpu.make_async_copy(gb1, agg_sh.at[dchunk.at[2]], sem02).wait()
        _unpack_dst(2, j0)
        pltpu.async_copy(gb1, agg_sh.at[dchunk.at[2]], sem02, add=True)
        pltpu.make_async_copy(gb1, agg_sh.at[dchunk.at[3]], sem03).wait()
        _unpack_dst(3, j0 + 1)
        pltpu.async_copy(gb1, agg_sh.at[dchunk.at[3]], sem03, add=True)
        return carry

    lax.fori_loop(1, (NCH - 1) // 2, cpair, 0)
    pltpu.make_async_copy(gb1, agg_sh.at[dchunk.at[2]], sem02).wait()
    _unpack_dst(2, NCH - 1)
    pltpu.async_copy(gb1, agg_sh.at[dchunk.at[2]], sem02, add=True)
    pltpu.make_async_copy(gb1, agg_sh.at[dchunk.at[3]], sem03).wait()
    pltpu.make_async_copy(gb1, agg_sh.at[dchunk.at[2]], sem02).wait()
    plsc.subcore_barrier()
    pltpu.sync_copy(agg_sh.at[pl.ds(rb, RPT)],
                    cntp_hbm.at[cid, pl.ds(rb, RPT)])
    pltpu.sync_copy(zrow_hbm, agg_sh.at[pl.ds(rb, RPT)])
    plsc.subcore_barrier()

    def pair(p, carry):
        j0 = 2 * p
        _unpack_chunk(pk_v, schunk, dchunk, 1, j0 + 1)
        _issue_gather(x_hbm, schunk, 1, gb1, sems1)
        _wait_gather(x_hbm, schunk, 0, gb0, sems0)
        pltpu.sync_copy(gb0, agg_sh.at[dchunk.at[0]], add=True)
        _unpack_chunk(pk_v, schunk, dchunk, 0, j0 + 2)
        _issue_gather(x_hbm, schunk, 0, gb0, sems0)
        _wait_gather(x_hbm, schunk, 1, gb1, sems1)
        pltpu.sync_copy(gb1, agg_sh.at[dchunk.at[1]], add=True)
        return carry

    lax.fori_loop(0, (NCH - 1) // 2, pair, 0)
    _wait_gather(x_hbm, schunk, 0, gb0, sems0)
    pltpu.sync_copy(gb0, agg_sh.at[dchunk.at[0]], add=True)
    plsc.subcore_barrier()

    pltpu.sync_copy(agg_sh.at[pl.ds(rb, RPT)],
                    aggp_hbm.at[cid, pl.ds(rb, RPT)])


_sc_agg_cnt = pl.kernel(
    _sc_agg_cnt_body,
    out_type=[jax.ShapeDtypeStruct((NC, NP, H), jnp.float32),
              jax.ShapeDtypeStruct((NC, NP, CW), jnp.float32)],
    mesh=plsc.VectorSubcoreMesh(**_MESH),
    scratch_types=[
        pltpu.VMEM((NCHP,), jnp.int32),
        pltpu.VMEM((NCHP, K), jnp.int32),
        pltpu.VMEM((2, K), jnp.int32),
        pltpu.VMEM((4, K), jnp.int32),
        pltpu.VMEM((K, H), jnp.float32),
        pltpu.VMEM((K, H), jnp.float32),
        pltpu.SemaphoreType.DMA,
        pltpu.SemaphoreType.DMA,
        pltpu.SemaphoreType.DMA,
        pltpu.SemaphoreType.DMA,
        pltpu.SemaphoreType.DMA,
        pltpu.SemaphoreType.DMA,
        pltpu.SemaphoreType.DMA,
        pltpu.SemaphoreType.DMA,
        pltpu.VMEM_SHARED((NP, H), jnp.float32),
    ],
)

BR = 1000


def _tc_body(aggp_ref, cntp_ref, x_ref, wl_ref, b_ref, wr_ref, o_ref):
    a = aggp_ref[0] + aggp_ref[1]
    c = jnp.maximum(cntp_ref[0, :, 0] + cntp_ref[1, :, 0], 1.0)
    agg = a / c[:, None]
    h = (jnp.dot(agg, wl_ref[...], preferred_element_type=jnp.float32)
         + b_ref[...]
         + jnp.dot(x_ref[...], wr_ref[...], preferred_element_type=jnp.float32))
    o_ref[...] = jnp.maximum(h, 0.0)


def _tc_layer(aggp, cntp, x, wl_t, b, wr_t):
    return pl.pallas_call(
        _tc_body,
        grid=(N // BR,),
        in_specs=[
            pl.BlockSpec((NC, BR, H), lambda i: (0, i, 0)),
            pl.BlockSpec((NC, BR, CW), lambda i: (0, i, 0)),
            pl.BlockSpec((BR, H), lambda i: (i, 0)),
            pl.BlockSpec((H, H), lambda i: (0, 0)),
            pl.BlockSpec((1, H), lambda i: (0, 0)),
            pl.BlockSpec((H, H), lambda i: (0, 0)),
        ],
        out_specs=pl.BlockSpec((BR, H), lambda i: (i, 0)),
        out_shape=jax.ShapeDtypeStruct((N, H), jnp.float32),
    )(aggp, cntp, x, wl_t, b, wr_t)


def _pack_idx(edge_index):
    src = edge_index[0].reshape(NW, EW)
    dst = edge_index[1].reshape(NW, EW)
    src = jnp.pad(src, ((0, 0), (0, EWP - EW)), constant_values=0)
    dst = jnp.pad(dst, ((0, 0), (0, EWP - EW)), constant_values=N)
    pk = jnp.left_shift(src, 16) | dst
    pk = pk.reshape(NW, NCH, K)
    pk = jnp.pad(pk, ((0, 0), (0, NCHP - NCH), (0, 0)))
    return pk.reshape(NW * NCHP, K)


def kernel(edge_index, user_emb, item_emb, W1_l, b1, W1_r, W2_l, b2, W2_r):
    x = jnp.concatenate([user_emb, item_emb], axis=0)
    pk = _pack_idx(edge_index)
    zrow = jnp.zeros((RPT, H), jnp.float32)
    ones = jnp.ones((K, H), jnp.float32)

    aggp1, cntp = _sc_agg_cnt(x, pk, zrow, ones)
    h1 = _tc_layer(aggp1, cntp, x, W1_l.T, b1.reshape(1, H), W1_r.T)
    (aggp2,) = _sc_agg(h1, pk, zrow)
    h2 = _tc_layer(aggp2, cntp, h1, W2_l.T, b2.reshape(1, H), W2_r.T)
    return h2[:NUM_U], h2[NUM_U:]
